# Initial kernel scaffold; baseline (speedup 1.0000x reference)
import jax
import jax.numpy as jnp
from jax.experimental import pallas as pl


def kernel(latent, edge_index_0, edge_index_1, edge_index_2, pos_0, pos_1, pos_2, params):
    raise NotImplementedError("write your pallas kernel here")


# speedup vs baseline: 3.0204x; 3.0204x over previous
import functools

import jax
import jax.numpy as jnp
from jax import lax
from jax.experimental import pallas as pl
from jax.experimental.pallas import tpu as pltpu
from jax.experimental.pallas import tpu_sc as plsc

_N0, _N1, _N2 = 2500, 5000, 10000
_H = 64
_LAT = 128

_NW = 32
_CHUNK = 128


def _matvec_head(latent, w0, b0, w1, b1):
    ncols = w1.shape[1]
    bc = 6400
    grid = ncols // bc

    def body(lat_ref, w0_ref, b0_ref, w1_ref, b1_ref, o_ref):
        v = jnp.maximum(
            jnp.dot(lat_ref[...], w0_ref[...],
                    preferred_element_type=jnp.float32) + b0_ref[...], 0.0)
        o_ref[...] = jnp.maximum(
            jnp.dot(v, w1_ref[...],
                    preferred_element_type=jnp.float32) + b1_ref[...], 0.0)

    out = pl.pallas_call(
        body,
        grid=(grid,),
        in_specs=[
            pl.BlockSpec((1, _LAT), lambda i: (0, 0)),
            pl.BlockSpec((_LAT, _LAT), lambda i: (0, 0)),
            pl.BlockSpec((1, _LAT), lambda i: (0, 0)),
            pl.BlockSpec((_LAT, bc), lambda i: (0, i)),
            pl.BlockSpec((1, bc), lambda i: (0, i)),
        ],
        out_specs=pl.BlockSpec((1, bc), lambda i: (0, i)),
        out_shape=jax.ShapeDtypeStruct((1, ncols), jnp.float32),
    )(latent.reshape(1, _LAT), w0, b0.reshape(1, _LAT),
      w1, b1.reshape(1, ncols))
    return out.reshape(_N0, _H)


@functools.partial(jax.jit, static_argnums=(4, 5))
def _segment_sum_sc(x, src3, dst3, zeros_tile, rows_acc, nchunk):
    rpt = rows_acc // 16
    mesh = plsc.VectorSubcoreMesh(core_axis_name="c", subcore_axis_name="s")

    @functools.partial(
        pl.kernel,
        out_type=jax.ShapeDtypeStruct((2, rows_acc, _H), jnp.float32),
        mesh=mesh,
        scratch_types=[
            pltpu.VMEM((nchunk, _CHUNK), jnp.int32),
            pltpu.VMEM((nchunk, _CHUNK), jnp.int32),
            pltpu.VMEM((_CHUNK, _H), jnp.float32),
            pltpu.VMEM((rpt, _H), jnp.float32),
            pltpu.VMEM_SHARED((rows_acc, _H), jnp.float32),
            pltpu.SemaphoreType.DMA,
        ],
        compiler_params=pltpu.CompilerParams(use_tc_tiling_on_sc=False),
    )
    def kern(x_hbm, src_hbm, dst_hbm, z_hbm, out_hbm,
             src_v, dst_v, rows_v, stage_v, acc_sh, sem):
        cid = lax.axis_index("c")
        sid = lax.axis_index("s")
        wid = sid * 2 + cid

        pltpu.sync_copy(z_hbm, stage_v)
        pltpu.sync_copy(stage_v, acc_sh.at[pl.ds(sid * rpt, rpt)])
        plsc.subcore_barrier()

        pltpu.sync_copy(src_hbm.at[wid], src_v)
        pltpu.sync_copy(dst_hbm.at[wid], dst_v)

        def body(j, carry):
            pltpu.async_copy(x_hbm.at[src_v.at[j]], rows_v, sem).wait()
            pltpu.sync_copy(rows_v, acc_sh.at[dst_v.at[j]], add=True)
            return carry

        lax.fori_loop(0, nchunk, body, 0, unroll=False)
        plsc.subcore_barrier()

        pltpu.sync_copy(acc_sh.at[pl.ds(sid * rpt, rpt)], stage_v)
        pltpu.sync_copy(stage_v, out_hbm.at[cid, pl.ds(sid * rpt, rpt)])

    return kern(x, src3, dst3, zeros_tile)


def _pad_edges(ei, n_dummy_dst, nchunk):
    e = ei.shape[1]
    tot = _NW * nchunk * _CHUNK
    pad = tot - e
    src = jnp.concatenate([ei[0], jnp.zeros((pad,), jnp.int32)])
    dst = jnp.concatenate([ei[1], jnp.full((pad,), n_dummy_dst, jnp.int32)])
    return (src.reshape(_NW, nchunk, _CHUNK),
            dst.reshape(_NW, nchunk, _CHUNK))


def _gin_mlp_tc(x, p0, p1, mp, relu_out):
    n = x.shape[0]
    chid = mp["lin0"]["W"].shape[1]
    cout = mp["lin1"]["W"].shape[1]
    br = 256
    grid = pl.cdiv(n, br)

    def body(x_ref, p0_ref, p1_ref,
             w0_ref, b0_ref, w1_ref, b1_ref,
             w2_ref, b2_ref, w3_ref, b3_ref, o_ref):
        h = x_ref[...] + p0_ref[...] + p1_ref[...]
        o1 = jnp.maximum(
            jnp.dot(h, w0_ref[...], preferred_element_type=jnp.float32)
            + b0_ref[...], 0.0)
        o1 = jnp.dot(o1, w1_ref[...],
                     preferred_element_type=jnp.float32) + b1_ref[...]
        o = jnp.maximum(
            jnp.dot(o1, w2_ref[...], preferred_element_type=jnp.float32)
            + b2_ref[...], 0.0)
        o = jnp.dot(o, w3_ref[...],
                    preferred_element_type=jnp.float32) + b3_ref[...] + o1
        if relu_out:
            o = jnp.maximum(o, 0.0)
        o_ref[...] = o

    full = lambda r, c: pl.BlockSpec((r, c), lambda i: (0, 0))
    out = pl.pallas_call(
        body,
        grid=(grid,),
        in_specs=[
            pl.BlockSpec((br, _H), lambda i: (i, 0)),
            pl.BlockSpec((br, _H), lambda i: (i, 0)),
            pl.BlockSpec((br, _H), lambda i: (i, 0)),
            full(_H, chid), full(1, chid),
            full(chid, cout), full(1, cout),
            full(cout, chid), full(1, chid),
            full(chid, cout), full(1, cout),
        ],
        out_specs=pl.BlockSpec((br, cout), lambda i: (i, 0)),
        out_shape=jax.ShapeDtypeStruct((n, cout), jnp.float32),
    )(x, p0, p1,
      mp["lin0"]["W"], mp["lin0"]["b"].reshape(1, -1),
      mp["lin1"]["W"], mp["lin1"]["b"].reshape(1, -1),
      mp["lin2"]["W"], mp["lin2"]["b"].reshape(1, -1),
      mp["lin3"]["W"], mp["lin3"]["b"].reshape(1, -1))
    return out


def _gin_conv(x, src3, dst3, zeros_tile, rows_acc, nchunk, mp, relu_out):
    n = x.shape[0]
    partials = _segment_sum_sc(x, src3, dst3, zeros_tile, rows_acc, nchunk)
    return _gin_mlp_tc(x, partials[0, :n], partials[1, :n], mp, relu_out)


def _knn_interpolate_tc(x, pos_y, pos_xt, k):
    ny = pos_y.shape[0]
    nxp = pos_xt.shape[1]
    h = x.shape[1]
    by = 256
    tile = 512
    nt = nxp // tile
    grid = pl.cdiv(ny, by)
    jbits = max(1, (nxp - 1).bit_length())

    def body(py_ref, pxt_ref, x_ref, o_ref, d_ref):
        yb = py_ref[...]
        y0 = yb[:, 0:1]
        y1 = yb[:, 1:2]
        y2 = yb[:, 2:3]

        def dist_body(t, carry):
            sl = pl.ds(t * tile, tile)
            x0 = pxt_ref[0:1, sl]
            x1 = pxt_ref[1:2, sl]
            x2 = pxt_ref[2:3, sl]
            d = (y0 - x0) ** 2 + (y1 - x1) ** 2 + (y2 - x2) ** 2
            d_ref[:, sl] = d
            return carry

        lax.fori_loop(0, nt, dist_body, 0, unroll=False)

        def count_le(mid):
            def tb(t, acc):
                d = d_ref[:, pl.ds(t * tile, tile)]
                db = lax.bitcast_convert_type(d, jnp.int32)
                return acc + jnp.sum((db <= mid).astype(jnp.int32),
                                     axis=1, keepdims=True)
            return lax.fori_loop(0, nt, tb, jnp.zeros((by, 1), jnp.int32),
                                 unroll=False)

        def bs_body(_, lohi):
            lo, hi = lohi
            mid = lax.div(lo + hi, 2)
            c = count_le(mid)
            ge = c >= k
            return (jnp.where(ge, lo, mid + 1), jnp.where(ge, mid, hi))

        lo0 = jnp.zeros((by, 1), jnp.int32)
        hi0 = jnp.full((by, 1), jnp.int32(0x7F800000))
        lo, hi = lax.fori_loop(0, 32, bs_body, (lo0, hi0), unroll=False)
        thresh = hi

        c_lt = count_le(thresh - 1)
        r = k - c_lt

        def count_eq_le(jcol):
            def tb(t, acc):
                d = d_ref[:, pl.ds(t * tile, tile)]
                db = lax.bitcast_convert_type(d, jnp.int32)
                col = lax.broadcasted_iota(jnp.int32, (by, tile), 1) + t * tile
                m = (db == thresh) & (col <= jcol)
                return acc + jnp.sum(m.astype(jnp.int32),
                                     axis=1, keepdims=True)
            return lax.fori_loop(0, nt, tb, jnp.zeros((by, 1), jnp.int32),
                                 unroll=False)

        def bs2_body(_, lohi):
            lo, hi = lohi
            mid = lax.div(lo + hi, 2)
            ge = count_eq_le(mid) >= r
            return (jnp.where(ge, lo, mid + 1), jnp.where(ge, mid, hi))

        lo0 = jnp.zeros((by, 1), jnp.int32)
        hi0 = jnp.full((by, 1), jnp.int32(nxp - 1))
        _, jstar = lax.fori_loop(0, jbits, bs2_body, (lo0, hi0),
                                 unroll=False)

        def w_body(t, carry):
            sl = pl.ds(t * tile, tile)
            d = d_ref[:, sl]
            db = lax.bitcast_convert_type(d, jnp.int32)
            col = lax.broadcasted_iota(jnp.int32, (by, tile), 1) + t * tile
            sel = (db < thresh) | ((db == thresh) & (col <= jstar))
            w = jnp.where(sel, 1.0 / jnp.maximum(d, 1e-16), 0.0)
            d_ref[:, sl] = w
            return carry

        lax.fori_loop(0, nt, w_body, 0, unroll=False)

        w = d_ref[...]
        num = jnp.dot(w, x_ref[...], preferred_element_type=jnp.float32)
        den = jnp.sum(w, axis=1, keepdims=True)
        o_ref[...] = num / den

    out = pl.pallas_call(
        body,
        grid=(grid,),
        in_specs=[
            pl.BlockSpec((by, 3), lambda i: (i, 0)),
            pl.BlockSpec((3, nxp), lambda i: (0, 0)),
            pl.BlockSpec((nxp, h), lambda i: (0, 0)),
        ],
        out_specs=pl.BlockSpec((by, h), lambda i: (i, 0)),
        out_shape=jax.ShapeDtypeStruct((ny, h), jnp.float32),
        scratch_shapes=[pltpu.VMEM((by, nxp), jnp.float32)],
    )(pos_y, pos_xt, x)
    return out


def _pad_pos(pos, nxp):
    n = pos.shape[0]
    pt = jnp.transpose(pos)
    return jnp.pad(pt, ((0, 0), (0, nxp - n)), constant_values=1e9)


def kernel(latent, edge_index_0, edge_index_1, edge_index_2,
           pos_0, pos_1, pos_2, params):
    p = params

    rows0, nch0 = 2560, 20
    rows1, nch1 = 5120, 40
    rows2, nch2 = 10112, 80
    z0 = jnp.zeros((rows0 // 16, _H), jnp.float32)
    z1 = jnp.zeros((rows1 // 16, _H), jnp.float32)
    z2 = jnp.zeros((rows2 // 16, _H), jnp.float32)
    s0, d0 = _pad_edges(edge_index_0, _N0, nch0)
    s1, d1 = _pad_edges(edge_index_1, _N1, nch1)
    s2, d2 = _pad_edges(edge_index_2, _N2, nch2)

    x = _matvec_head(latent, p["lin_0"]["W"], p["lin_0"]["b"],
                     p["lin_1"]["W"], p["lin_1"]["b"])

    x = _gin_conv(x, s0, d0, z0, rows0, nch0, p["conv0"], True)
    x = _gin_conv(x, s0, d0, z0, rows0, nch0, p["conv1"], True)

    nxp0 = 2560
    xp = jnp.pad(x, ((0, nxp0 - _N0), (0, 0)))
    x = _knn_interpolate_tc(xp, pos_1, _pad_pos(pos_0, nxp0), 32)

    x = _gin_conv(x, s1, d1, z1, rows1, nch1, p["conv2"], True)

    nxp1 = 5120
    xp = jnp.pad(x, ((0, nxp1 - _N1), (0, 0)))
    x = _knn_interpolate_tc(xp, pos_2, _pad_pos(pos_1, nxp1), 32)

    x = _gin_conv(x, s2, d2, z2, rows2, nch2, p["conv3"], True)
    out = _gin_conv(x, s2, d2, z2, rows2, nch2, p["conv4"], False)
    return out


# speedup vs baseline: 4.3591x; 1.4432x over previous
import functools

import jax
import jax.numpy as jnp
from jax import lax
from jax.experimental import pallas as pl
from jax.experimental.pallas import tpu as pltpu
from jax.experimental.pallas import tpu_sc as plsc

_N0, _N1, _N2 = 2500, 5000, 10000
_H = 64
_LAT = 128

_NW = 32
_CHUNK = 128


def _matvec_head(latent, w0, b0, w1, b1):
    ncols = w1.shape[1]
    bc = 6400
    grid = ncols // bc

    def body(lat_ref, w0_ref, b0_ref, w1_ref, b1_ref, o_ref):
        v = jnp.maximum(
            jnp.dot(lat_ref[...], w0_ref[...],
                    preferred_element_type=jnp.float32) + b0_ref[...], 0.0)
        o_ref[...] = jnp.maximum(
            jnp.dot(v, w1_ref[...],
                    preferred_element_type=jnp.float32) + b1_ref[...], 0.0)

    out = pl.pallas_call(
        body,
        grid=(grid,),
        in_specs=[
            pl.BlockSpec((1, _LAT), lambda i: (0, 0)),
            pl.BlockSpec((_LAT, _LAT), lambda i: (0, 0)),
            pl.BlockSpec((1, _LAT), lambda i: (0, 0)),
            pl.BlockSpec((_LAT, bc), lambda i: (0, i)),
            pl.BlockSpec((1, bc), lambda i: (0, i)),
        ],
        out_specs=pl.BlockSpec((1, bc), lambda i: (0, i)),
        out_shape=jax.ShapeDtypeStruct((1, ncols), jnp.float32),
    )(latent.reshape(1, _LAT), w0, b0.reshape(1, _LAT),
      w1, b1.reshape(1, ncols))
    return out.reshape(_N0, _H)


@functools.partial(jax.jit, static_argnums=(4, 5))
def _segment_sum_sc(x, src3, dst3, zeros_tile, rows_acc, nchunk):
    rpt = rows_acc // 16
    mesh = plsc.VectorSubcoreMesh(core_axis_name="c", subcore_axis_name="s")

    @functools.partial(
        pl.kernel,
        out_type=jax.ShapeDtypeStruct((2, rows_acc, _H), jnp.float32),
        mesh=mesh,
        scratch_types=[
            pltpu.VMEM((nchunk, _CHUNK), jnp.int32),
            pltpu.VMEM((nchunk, _CHUNK), jnp.int32),
            pltpu.VMEM((_CHUNK, _H), jnp.float32),
            pltpu.VMEM((rpt, _H), jnp.float32),
            pltpu.VMEM_SHARED((rows_acc, _H), jnp.float32),
            pltpu.SemaphoreType.DMA,
        ],
        compiler_params=pltpu.CompilerParams(use_tc_tiling_on_sc=False),
    )
    def kern(x_hbm, src_hbm, dst_hbm, z_hbm, out_hbm,
             src_v, dst_v, rows_v, stage_v, acc_sh, sem):
        cid = lax.axis_index("c")
        sid = lax.axis_index("s")
        wid = sid * 2 + cid

        pltpu.sync_copy(z_hbm, stage_v)
        pltpu.sync_copy(stage_v, acc_sh.at[pl.ds(sid * rpt, rpt)])
        plsc.subcore_barrier()

        pltpu.sync_copy(src_hbm.at[wid], src_v)
        pltpu.sync_copy(dst_hbm.at[wid], dst_v)

        def body(j, carry):
            pltpu.async_copy(x_hbm.at[src_v.at[j]], rows_v, sem).wait()
            pltpu.sync_copy(rows_v, acc_sh.at[dst_v.at[j]], add=True)
            return carry

        lax.fori_loop(0, nchunk, body, 0, unroll=False)
        plsc.subcore_barrier()

        pltpu.sync_copy(acc_sh.at[pl.ds(sid * rpt, rpt)], stage_v)
        pltpu.sync_copy(stage_v, out_hbm.at[cid, pl.ds(sid * rpt, rpt)])

    return kern(x, src3, dst3, zeros_tile)


def _pad_edges(ei, n_dummy_dst, nchunk):
    e = ei.shape[1]
    tot = _NW * nchunk * _CHUNK
    pad = tot - e
    src = jnp.concatenate([ei[0], jnp.zeros((pad,), jnp.int32)])
    dst = jnp.concatenate([ei[1], jnp.full((pad,), n_dummy_dst, jnp.int32)])
    return (src.reshape(_NW, nchunk, _CHUNK),
            dst.reshape(_NW, nchunk, _CHUNK))


def _gin_mlp_tc(x, p0, p1, mp, relu_out):
    n = x.shape[0]
    chid = mp["lin0"]["W"].shape[1]
    cout = mp["lin1"]["W"].shape[1]
    br = 256
    grid = pl.cdiv(n, br)

    def body(x_ref, p0_ref, p1_ref,
             w0_ref, b0_ref, w1_ref, b1_ref,
             w2_ref, b2_ref, w3_ref, b3_ref, o_ref):
        h = x_ref[...] + p0_ref[...] + p1_ref[...]
        o1 = jnp.maximum(
            jnp.dot(h, w0_ref[...], preferred_element_type=jnp.float32)
            + b0_ref[...], 0.0)
        o1 = jnp.dot(o1, w1_ref[...],
                     preferred_element_type=jnp.float32) + b1_ref[...]
        o = jnp.maximum(
            jnp.dot(o1, w2_ref[...], preferred_element_type=jnp.float32)
            + b2_ref[...], 0.0)
        o = jnp.dot(o, w3_ref[...],
                    preferred_element_type=jnp.float32) + b3_ref[...] + o1
        if relu_out:
            o = jnp.maximum(o, 0.0)
        o_ref[...] = o

    full = lambda r, c: pl.BlockSpec((r, c), lambda i: (0, 0))
    out = pl.pallas_call(
        body,
        grid=(grid,),
        in_specs=[
            pl.BlockSpec((br, _H), lambda i: (i, 0)),
            pl.BlockSpec((br, _H), lambda i: (i, 0)),
            pl.BlockSpec((br, _H), lambda i: (i, 0)),
            full(_H, chid), full(1, chid),
            full(chid, cout), full(1, cout),
            full(cout, chid), full(1, chid),
            full(chid, cout), full(1, cout),
        ],
        out_specs=pl.BlockSpec((br, cout), lambda i: (i, 0)),
        out_shape=jax.ShapeDtypeStruct((n, cout), jnp.float32),
    )(x, p0, p1,
      mp["lin0"]["W"], mp["lin0"]["b"].reshape(1, -1),
      mp["lin1"]["W"], mp["lin1"]["b"].reshape(1, -1),
      mp["lin2"]["W"], mp["lin2"]["b"].reshape(1, -1),
      mp["lin3"]["W"], mp["lin3"]["b"].reshape(1, -1))
    return out


def _gin_conv(x, src3, dst3, zeros_tile, rows_acc, nchunk, mp, relu_out):
    n = x.shape[0]
    partials = _segment_sum_sc(x, src3, dst3, zeros_tile, rows_acc, nchunk)
    return _gin_mlp_tc(x, partials[0, :n], partials[1, :n], mp, relu_out)


def _knn_interpolate_tc(x, pos_y, pos_xt, k):
    ny = pos_y.shape[0]
    nxp = pos_xt.shape[1]
    h = x.shape[1]
    by = 256
    tile = 512
    nt = nxp // tile
    grid = pl.cdiv(ny, by)
    jbits = max(1, (nxp - 1).bit_length())

    def body(py_ref, pxt_ref, x_ref, o_ref, d_ref):
        yb = py_ref[...]
        y0 = yb[:, 0:1]
        y1 = yb[:, 1:2]
        y2 = yb[:, 2:3]

        def dist_body(t, carry):
            sl = pl.ds(t * tile, tile)
            x0 = pxt_ref[0:1, sl]
            x1 = pxt_ref[1:2, sl]
            x2 = pxt_ref[2:3, sl]
            d = (y0 - x0) ** 2 + (y1 - x1) ** 2 + (y2 - x2) ** 2
            d_ref[:, sl] = d
            return carry

        lax.fori_loop(0, nt, dist_body, 0, unroll=False)

        nsl = tile // 128

        def _lane_fold(m):
            s = m[:, 0:128]
            for q in range(1, nsl):
                s = s + m[:, q * 128:(q + 1) * 128]
            return s

        def count_le(mid):
            def tb(t, acc):
                d = d_ref[:, pl.ds(t * tile, tile)]
                db = lax.bitcast_convert_type(d, jnp.int32)
                return acc + _lane_fold((db <= mid).astype(jnp.int32))
            acc = lax.fori_loop(0, nt, tb, jnp.zeros((by, 128), jnp.int32),
                                unroll=False)
            return jnp.sum(acc, axis=1, keepdims=True)

        def bs_body(_, lohi):
            lo, hi = lohi
            mid = lax.div(lo + hi, 2)
            c = count_le(mid)
            ge = c >= k
            return (jnp.where(ge, lo, mid + 1), jnp.where(ge, mid, hi))

        lo0 = jnp.zeros((by, 1), jnp.int32)
        hi0 = jnp.full((by, 1), jnp.int32(0x7F800000))
        lo, hi = lax.fori_loop(0, 31, bs_body, (lo0, hi0), unroll=False)
        thresh = hi

        c_lt = count_le(thresh - 1)
        r = k - c_lt

        def count_eq_le(jcol):
            def tb(t, acc):
                d = d_ref[:, pl.ds(t * tile, tile)]
                db = lax.bitcast_convert_type(d, jnp.int32)
                col = lax.broadcasted_iota(jnp.int32, (by, tile), 1) + t * tile
                m = (db == thresh) & (col <= jcol)
                return acc + _lane_fold(m.astype(jnp.int32))
            acc = lax.fori_loop(0, nt, tb, jnp.zeros((by, 128), jnp.int32),
                                unroll=False)
            return jnp.sum(acc, axis=1, keepdims=True)

        def bs2_body(_, lohi):
            lo, hi = lohi
            mid = lax.div(lo + hi, 2)
            ge = count_eq_le(mid) >= r
            return (jnp.where(ge, lo, mid + 1), jnp.where(ge, mid, hi))

        lo0 = jnp.zeros((by, 1), jnp.int32)
        hi0 = jnp.full((by, 1), jnp.int32(nxp - 1))
        _, jstar = lax.fori_loop(0, jbits, bs2_body, (lo0, hi0),
                                 unroll=False)

        def w_body(t, carry):
            sl = pl.ds(t * tile, tile)
            d = d_ref[:, sl]
            db = lax.bitcast_convert_type(d, jnp.int32)
            col = lax.broadcasted_iota(jnp.int32, (by, tile), 1) + t * tile
            sel = (db < thresh) | ((db == thresh) & (col <= jstar))
            w = jnp.where(sel, 1.0 / jnp.maximum(d, 1e-16), 0.0)
            d_ref[:, sl] = w
            return carry

        lax.fori_loop(0, nt, w_body, 0, unroll=False)

        w = d_ref[...]
        num = jnp.dot(w, x_ref[...], preferred_element_type=jnp.float32)
        den = jnp.sum(w, axis=1, keepdims=True)
        o_ref[...] = num / den

    out = pl.pallas_call(
        body,
        grid=(grid,),
        in_specs=[
            pl.BlockSpec((by, 3), lambda i: (i, 0)),
            pl.BlockSpec((3, nxp), lambda i: (0, 0)),
            pl.BlockSpec((nxp, h), lambda i: (0, 0)),
        ],
        out_specs=pl.BlockSpec((by, h), lambda i: (i, 0)),
        out_shape=jax.ShapeDtypeStruct((ny, h), jnp.float32),
        scratch_shapes=[pltpu.VMEM((by, nxp), jnp.float32)],
    )(pos_y, pos_xt, x)
    return out


def _pad_pos(pos, nxp):
    n = pos.shape[0]
    pt = jnp.transpose(pos)
    return jnp.pad(pt, ((0, 0), (0, nxp - n)), constant_values=1e9)


def kernel(latent, edge_index_0, edge_index_1, edge_index_2,
           pos_0, pos_1, pos_2, params):
    p = params

    rows0, nch0 = 2560, 20
    rows1, nch1 = 5120, 40
    rows2, nch2 = 10112, 80
    z0 = jnp.zeros((rows0 // 16, _H), jnp.float32)
    z1 = jnp.zeros((rows1 // 16, _H), jnp.float32)
    z2 = jnp.zeros((rows2 // 16, _H), jnp.float32)
    s0, d0 = _pad_edges(edge_index_0, _N0, nch0)
    s1, d1 = _pad_edges(edge_index_1, _N1, nch1)
    s2, d2 = _pad_edges(edge_index_2, _N2, nch2)

    x = _matvec_head(latent, p["lin_0"]["W"], p["lin_0"]["b"],
                     p["lin_1"]["W"], p["lin_1"]["b"])

    x = _gin_conv(x, s0, d0, z0, rows0, nch0, p["conv0"], True)
    x = _gin_conv(x, s0, d0, z0, rows0, nch0, p["conv1"], True)

    nxp0 = 2560
    xp = jnp.pad(x, ((0, nxp0 - _N0), (0, 0)))
    x = _knn_interpolate_tc(xp, pos_1, _pad_pos(pos_0, nxp0), 32)

    x = _gin_conv(x, s1, d1, z1, rows1, nch1, p["conv2"], True)

    nxp1 = 5120
    xp = jnp.pad(x, ((0, nxp1 - _N1), (0, 0)))
    x = _knn_interpolate_tc(xp, pos_2, _pad_pos(pos_1, nxp1), 32)

    x = _gin_conv(x, s2, d2, z2, rows2, nch2, p["conv3"], True)
    out = _gin_conv(x, s2, d2, z2, rows2, nch2, p["conv4"], False)
    return out


# speedup vs baseline: 4.5441x; 1.0425x over previous
import functools

import jax
import jax.numpy as jnp
from jax import lax
from jax.experimental import pallas as pl
from jax.experimental.pallas import tpu as pltpu
from jax.experimental.pallas import tpu_sc as plsc

_N0, _N1, _N2 = 2500, 5000, 10000
_H = 64
_LAT = 128

_NW = 32
_CHUNK = 128


def _matvec_head(latent, w0, b0, w1, b1):
    ncols = w1.shape[1]
    bc = 6400
    grid = ncols // bc

    def body(lat_ref, w0_ref, b0_ref, w1_ref, b1_ref, o_ref):
        v = jnp.maximum(
            jnp.dot(lat_ref[...], w0_ref[...],
                    preferred_element_type=jnp.float32) + b0_ref[...], 0.0)
        o_ref[...] = jnp.maximum(
            jnp.dot(v, w1_ref[...],
                    preferred_element_type=jnp.float32) + b1_ref[...], 0.0)

    out = pl.pallas_call(
        body,
        grid=(grid,),
        in_specs=[
            pl.BlockSpec((1, _LAT), lambda i: (0, 0)),
            pl.BlockSpec((_LAT, _LAT), lambda i: (0, 0)),
            pl.BlockSpec((1, _LAT), lambda i: (0, 0)),
            pl.BlockSpec((_LAT, bc), lambda i: (0, i)),
            pl.BlockSpec((1, bc), lambda i: (0, i)),
        ],
        out_specs=pl.BlockSpec((1, bc), lambda i: (0, i)),
        out_shape=jax.ShapeDtypeStruct((1, ncols), jnp.float32),
    )(latent.reshape(1, _LAT), w0, b0.reshape(1, _LAT),
      w1, b1.reshape(1, ncols))
    return out.reshape(_N0, _H)


@functools.partial(jax.jit, static_argnums=(4, 5, 6))
def _segment_sum_sc(x, src3, dst3, zeros_tile, rows_acc, nchunk, gk):
    rpt = rows_acc // 16
    ngroups = nchunk // gk
    gsz = gk * _CHUNK
    mesh = plsc.VectorSubcoreMesh(core_axis_name="c", subcore_axis_name="s")

    @functools.partial(
        pl.kernel,
        out_type=jax.ShapeDtypeStruct((2, rows_acc, _H), jnp.float32),
        mesh=mesh,
        scratch_types=[
            pltpu.VMEM((nchunk, _CHUNK), jnp.int32),
            pltpu.VMEM((nchunk, _CHUNK), jnp.int32),
            pltpu.VMEM((2, gsz, _H), jnp.float32),
            pltpu.VMEM_SHARED((rows_acc, _H), jnp.float32),
            pltpu.SemaphoreType.DMA,
            pltpu.SemaphoreType.DMA,
        ],
        compiler_params=pltpu.CompilerParams(use_tc_tiling_on_sc=False),
    )
    def kern(x_hbm, src_hbm, dst_hbm, z_hbm, out_hbm,
             src_v, dst_v, rows_v, acc_sh, sem0, sem1):
        cid = lax.axis_index("c")
        sid = lax.axis_index("s")
        wid = sid * 2 + cid
        sems = (sem0, sem1)
        pieces = [(o, min(gsz, rpt - o)) for o in range(0, rpt, gsz)]

        for off, sz in pieces:
            pltpu.sync_copy(z_hbm.at[pl.ds(off, sz)],
                            rows_v.at[0, pl.ds(0, sz)])
            pltpu.sync_copy(rows_v.at[0, pl.ds(0, sz)],
                            acc_sh.at[pl.ds(sid * rpt + off, sz)])
        plsc.subcore_barrier()

        pltpu.sync_copy(src_hbm.at[wid], src_v)
        pltpu.sync_copy(dst_hbm.at[wid], dst_v)

        def fire(g, b):
            for c in range(gk):
                pltpu.async_copy(x_hbm.at[src_v.at[g * gk + c]],
                                 rows_v.at[b, pl.ds(c * _CHUNK, _CHUNK)],
                                 sems[b])

        def drain_scatter(g, b):
            pltpu.make_async_copy(x_hbm.at[pl.ds(0, gsz)],
                                  rows_v.at[b], sems[b]).wait()
            for c in range(gk):
                pltpu.sync_copy(rows_v.at[b, pl.ds(c * _CHUNK, _CHUNK)],
                                acc_sh.at[dst_v.at[g * gk + c]], add=True)

        fire(0, 0)

        def body(gp, carry):
            g0 = 2 * gp
            fire(g0 + 1, 1)
            drain_scatter(g0, 0)

            @pl.when(gp < ngroups // 2 - 1)
            def _():
                fire(g0 + 2, 0)

            drain_scatter(g0 + 1, 1)
            return carry

        lax.fori_loop(0, ngroups // 2, body, 0, unroll=False)
        plsc.subcore_barrier()

        for off, sz in pieces:
            pltpu.sync_copy(acc_sh.at[pl.ds(sid * rpt + off, sz)],
                            rows_v.at[0, pl.ds(0, sz)])
            pltpu.sync_copy(rows_v.at[0, pl.ds(0, sz)],
                            out_hbm.at[cid, pl.ds(sid * rpt + off, sz)])

    return kern(x, src3, dst3, zeros_tile)


def _pad_edges(ei, n_dummy_dst, nchunk):
    e = ei.shape[1]
    tot = _NW * nchunk * _CHUNK
    pad = tot - e
    src = jnp.concatenate([ei[0], jnp.zeros((pad,), jnp.int32)])
    dst = jnp.concatenate([ei[1], jnp.full((pad,), n_dummy_dst, jnp.int32)])
    return (src.reshape(_NW, nchunk, _CHUNK),
            dst.reshape(_NW, nchunk, _CHUNK))


def _gin_mlp_tc(x, p0, p1, mp, relu_out):
    n = x.shape[0]
    chid = mp["lin0"]["W"].shape[1]
    cout = mp["lin1"]["W"].shape[1]
    br = 256
    grid = pl.cdiv(n, br)

    def body(x_ref, p0_ref, p1_ref,
             w0_ref, b0_ref, w1_ref, b1_ref,
             w2_ref, b2_ref, w3_ref, b3_ref, o_ref):
        h = x_ref[...] + p0_ref[...] + p1_ref[...]
        o1 = jnp.maximum(
            jnp.dot(h, w0_ref[...], preferred_element_type=jnp.float32)
            + b0_ref[...], 0.0)
        o1 = jnp.dot(o1, w1_ref[...],
                     preferred_element_type=jnp.float32) + b1_ref[...]
        o = jnp.maximum(
            jnp.dot(o1, w2_ref[...], preferred_element_type=jnp.float32)
            + b2_ref[...], 0.0)
        o = jnp.dot(o, w3_ref[...],
                    preferred_element_type=jnp.float32) + b3_ref[...] + o1
        if relu_out:
            o = jnp.maximum(o, 0.0)
        o_ref[...] = o

    full = lambda r, c: pl.BlockSpec((r, c), lambda i: (0, 0))
    out = pl.pallas_call(
        body,
        grid=(grid,),
        in_specs=[
            pl.BlockSpec((br, _H), lambda i: (i, 0)),
            pl.BlockSpec((br, _H), lambda i: (i, 0)),
            pl.BlockSpec((br, _H), lambda i: (i, 0)),
            full(_H, chid), full(1, chid),
            full(chid, cout), full(1, cout),
            full(cout, chid), full(1, chid),
            full(chid, cout), full(1, cout),
        ],
        out_specs=pl.BlockSpec((br, cout), lambda i: (i, 0)),
        out_shape=jax.ShapeDtypeStruct((n, cout), jnp.float32),
    )(x, p0, p1,
      mp["lin0"]["W"], mp["lin0"]["b"].reshape(1, -1),
      mp["lin1"]["W"], mp["lin1"]["b"].reshape(1, -1),
      mp["lin2"]["W"], mp["lin2"]["b"].reshape(1, -1),
      mp["lin3"]["W"], mp["lin3"]["b"].reshape(1, -1))
    return out


def _gin_conv(x, src3, dst3, zeros_tile, rows_acc, nchunk, gk, mp, relu_out):
    n = x.shape[0]
    partials = _segment_sum_sc(x, src3, dst3, zeros_tile, rows_acc, nchunk, gk)
    return _gin_mlp_tc(x, partials[0, :n], partials[1, :n], mp, relu_out)


def _knn_interpolate_tc(x, pos_y, pos_xt, k):
    ny = pos_y.shape[0]
    nxp = pos_xt.shape[1]
    h = x.shape[1]
    by = 256
    tile = 512
    nt = nxp // tile
    grid = pl.cdiv(ny, by)
    jbits = max(1, (nxp - 1).bit_length())

    def body(py_ref, pxt_ref, x_ref, o_ref, d_ref):
        yb = py_ref[...]
        y0 = yb[:, 0:1]
        y1 = yb[:, 1:2]
        y2 = yb[:, 2:3]

        def dist_body(t, carry):
            sl = pl.ds(t * tile, tile)
            x0 = pxt_ref[0:1, sl]
            x1 = pxt_ref[1:2, sl]
            x2 = pxt_ref[2:3, sl]
            d = (y0 - x0) ** 2 + (y1 - x1) ** 2 + (y2 - x2) ** 2
            d_ref[:, sl] = d
            return carry

        lax.fori_loop(0, nt, dist_body, 0, unroll=False)

        nsl = tile // 128

        def _lane_fold(m):
            s = m[:, 0:128]
            for q in range(1, nsl):
                s = s + m[:, q * 128:(q + 1) * 128]
            return s

        def count_le(mid):
            def tb(t, acc):
                d = d_ref[:, pl.ds(t * tile, tile)]
                db = lax.bitcast_convert_type(d, jnp.int32)
                return acc + _lane_fold((db <= mid).astype(jnp.int32))
            acc = lax.fori_loop(0, nt, tb, jnp.zeros((by, 128), jnp.int32),
                                unroll=False)
            return jnp.sum(acc, axis=1, keepdims=True)

        def bs_body(_, lohi):
            lo, hi = lohi
            mid = lax.div(lo + hi, 2)
            c = count_le(mid)
            ge = c >= k
            return (jnp.where(ge, lo, mid + 1), jnp.where(ge, mid, hi))

        lo0 = jnp.zeros((by, 1), jnp.int32)
        hi0 = jnp.full((by, 1), jnp.int32(0x7F800000))
        lo, hi = lax.fori_loop(0, 31, bs_body, (lo0, hi0), unroll=False)
        thresh = hi

        c_lt = count_le(thresh - 1)
        r = k - c_lt

        def count_eq_le(jcol):
            def tb(t, acc):
                d = d_ref[:, pl.ds(t * tile, tile)]
                db = lax.bitcast_convert_type(d, jnp.int32)
                col = lax.broadcasted_iota(jnp.int32, (by, tile), 1) + t * tile
                m = (db == thresh) & (col <= jcol)
                return acc + _lane_fold(m.astype(jnp.int32))
            acc = lax.fori_loop(0, nt, tb, jnp.zeros((by, 128), jnp.int32),
                                unroll=False)
            return jnp.sum(acc, axis=1, keepdims=True)

        def bs2_body(_, lohi):
            lo, hi = lohi
            mid = lax.div(lo + hi, 2)
            ge = count_eq_le(mid) >= r
            return (jnp.where(ge, lo, mid + 1), jnp.where(ge, mid, hi))

        lo0 = jnp.zeros((by, 1), jnp.int32)
        hi0 = jnp.full((by, 1), jnp.int32(nxp - 1))
        _, jstar = lax.fori_loop(0, jbits, bs2_body, (lo0, hi0),
                                 unroll=False)

        def w_body(t, carry):
            sl = pl.ds(t * tile, tile)
            d = d_ref[:, sl]
            db = lax.bitcast_convert_type(d, jnp.int32)
            col = lax.broadcasted_iota(jnp.int32, (by, tile), 1) + t * tile
            sel = (db < thresh) | ((db == thresh) & (col <= jstar))
            w = jnp.where(sel, 1.0 / jnp.maximum(d, 1e-16), 0.0)
            d_ref[:, sl] = w
            return carry

        lax.fori_loop(0, nt, w_body, 0, unroll=False)

        w = d_ref[...]
        num = jnp.dot(w, x_ref[...], preferred_element_type=jnp.float32)
        den = jnp.sum(w, axis=1, keepdims=True)
        o_ref[...] = num / den

    out = pl.pallas_call(
        body,
        grid=(grid,),
        in_specs=[
            pl.BlockSpec((by, 3), lambda i: (i, 0)),
            pl.BlockSpec((3, nxp), lambda i: (0, 0)),
            pl.BlockSpec((nxp, h), lambda i: (0, 0)),
        ],
        out_specs=pl.BlockSpec((by, h), lambda i: (i, 0)),
        out_shape=jax.ShapeDtypeStruct((ny, h), jnp.float32),
        scratch_shapes=[pltpu.VMEM((by, nxp), jnp.float32)],
    )(pos_y, pos_xt, x)
    return out


def _pad_pos(pos, nxp):
    n = pos.shape[0]
    pt = jnp.transpose(pos)
    return jnp.pad(pt, ((0, 0), (0, nxp - n)), constant_values=1e9)


def kernel(latent, edge_index_0, edge_index_1, edge_index_2,
           pos_0, pos_1, pos_2, params):
    p = params

    rows0, nch0 = 2560, 20
    rows1, nch1 = 5120, 40
    rows2, nch2 = 10112, 80
    z0 = jnp.zeros((rows0 // 16, _H), jnp.float32)
    z1 = jnp.zeros((rows1 // 16, _H), jnp.float32)
    z2 = jnp.zeros((rows2 // 16, _H), jnp.float32)
    s0, d0 = _pad_edges(edge_index_0, _N0, nch0)
    s1, d1 = _pad_edges(edge_index_1, _N1, nch1)
    s2, d2 = _pad_edges(edge_index_2, _N2, nch2)

    x = _matvec_head(latent, p["lin_0"]["W"], p["lin_0"]["b"],
                     p["lin_1"]["W"], p["lin_1"]["b"])

    x = _gin_conv(x, s0, d0, z0, rows0, nch0, 2, p["conv0"], True)
    x = _gin_conv(x, s0, d0, z0, rows0, nch0, 2, p["conv1"], True)

    nxp0 = 2560
    xp = jnp.pad(x, ((0, nxp0 - _N0), (0, 0)))
    x = _knn_interpolate_tc(xp, pos_1, _pad_pos(pos_0, nxp0), 32)

    x = _gin_conv(x, s1, d1, z1, rows1, nch1, 4, p["conv2"], True)

    nxp1 = 5120
    xp = jnp.pad(x, ((0, nxp1 - _N1), (0, 0)))
    x = _knn_interpolate_tc(xp, pos_2, _pad_pos(pos_1, nxp1), 32)

    x = _gin_conv(x, s2, d2, z2, rows2, nch2, 4, p["conv3"], True)
    out = _gin_conv(x, s2, d2, z2, rows2, nch2, 4, p["conv4"], False)
    return out


# speedup vs baseline: 4.5519x; 1.0017x over previous
import functools

import jax
import jax.numpy as jnp
from jax import lax
from jax.experimental import pallas as pl
from jax.experimental.pallas import tpu as pltpu
from jax.experimental.pallas import tpu_sc as plsc

_N0, _N1, _N2 = 2500, 5000, 10000
_H = 64
_LAT = 128

_NW = 32
_CHUNK = 128


def _matvec_head(latent, w0, b0, w1, b1):
    ncols = w1.shape[1]
    bc = 6400
    grid = ncols // bc

    def body(lat_ref, w0_ref, b0_ref, w1_ref, b1_ref, o_ref):
        v = jnp.maximum(
            jnp.dot(lat_ref[...], w0_ref[...],
                    preferred_element_type=jnp.float32) + b0_ref[...], 0.0)
        o_ref[...] = jnp.maximum(
            jnp.dot(v, w1_ref[...],
                    preferred_element_type=jnp.float32) + b1_ref[...], 0.0)

    out = pl.pallas_call(
        body,
        grid=(grid,),
        in_specs=[
            pl.BlockSpec((1, _LAT), lambda i: (0, 0)),
            pl.BlockSpec((_LAT, _LAT), lambda i: (0, 0)),
            pl.BlockSpec((1, _LAT), lambda i: (0, 0)),
            pl.BlockSpec((_LAT, bc), lambda i: (0, i)),
            pl.BlockSpec((1, bc), lambda i: (0, i)),
        ],
        out_specs=pl.BlockSpec((1, bc), lambda i: (0, i)),
        out_shape=jax.ShapeDtypeStruct((1, ncols), jnp.float32),
    )(latent.reshape(1, _LAT), w0, b0.reshape(1, _LAT),
      w1, b1.reshape(1, ncols))
    return out.reshape(_N0, _H)


@functools.partial(jax.jit, static_argnums=(4, 5, 6))
def _segment_sum_sc(x, src3, dst3, zeros_tile, rows_acc, nchunk, gk):
    rpt = rows_acc // 16
    ngroups = nchunk // gk
    gsz = gk * _CHUNK
    mesh = plsc.VectorSubcoreMesh(core_axis_name="c", subcore_axis_name="s")

    @functools.partial(
        pl.kernel,
        out_type=jax.ShapeDtypeStruct((2, rows_acc, _H), jnp.float32),
        mesh=mesh,
        scratch_types=[
            pltpu.VMEM((nchunk, _CHUNK), jnp.int32),
            pltpu.VMEM((nchunk, _CHUNK), jnp.int32),
            pltpu.VMEM((2, gsz, _H), jnp.float32),
            pltpu.VMEM_SHARED((rows_acc, _H), jnp.float32),
            pltpu.SemaphoreType.DMA,
            pltpu.SemaphoreType.DMA,
        ],
        compiler_params=pltpu.CompilerParams(use_tc_tiling_on_sc=False),
    )
    def kern(x_hbm, src_hbm, dst_hbm, z_hbm, out_hbm,
             src_v, dst_v, rows_v, acc_sh, sem0, sem1):
        cid = lax.axis_index("c")
        sid = lax.axis_index("s")
        wid = sid * 2 + cid
        sems = (sem0, sem1)
        pieces = [(o, min(gsz, rpt - o)) for o in range(0, rpt, gsz)]

        for off, sz in pieces:
            pltpu.sync_copy(z_hbm.at[pl.ds(off, sz)],
                            rows_v.at[0, pl.ds(0, sz)])
            pltpu.sync_copy(rows_v.at[0, pl.ds(0, sz)],
                            acc_sh.at[pl.ds(sid * rpt + off, sz)])
        plsc.subcore_barrier()

        pltpu.sync_copy(src_hbm.at[wid], src_v)
        pltpu.sync_copy(dst_hbm.at[wid], dst_v)

        def fire(g, b):
            for c in range(gk):
                pltpu.async_copy(x_hbm.at[src_v.at[g * gk + c]],
                                 rows_v.at[b, pl.ds(c * _CHUNK, _CHUNK)],
                                 sems[b])

        def drain_scatter(g, b):
            pltpu.make_async_copy(x_hbm.at[pl.ds(0, gsz)],
                                  rows_v.at[b], sems[b]).wait()
            for c in range(gk):
                pltpu.sync_copy(rows_v.at[b, pl.ds(c * _CHUNK, _CHUNK)],
                                acc_sh.at[dst_v.at[g * gk + c]], add=True)

        fire(0, 0)

        def body(gp, carry):
            g0 = 2 * gp
            fire(g0 + 1, 1)
            drain_scatter(g0, 0)

            @pl.when(gp < ngroups // 2 - 1)
            def _():
                fire(g0 + 2, 0)

            drain_scatter(g0 + 1, 1)
            return carry

        lax.fori_loop(0, ngroups // 2, body, 0, unroll=False)
        plsc.subcore_barrier()

        for off, sz in pieces:
            pltpu.sync_copy(acc_sh.at[pl.ds(sid * rpt + off, sz)],
                            rows_v.at[0, pl.ds(0, sz)])
            pltpu.sync_copy(rows_v.at[0, pl.ds(0, sz)],
                            out_hbm.at[cid, pl.ds(sid * rpt + off, sz)])

    return kern(x, src3, dst3, zeros_tile)


def _pad_edges(ei, n_dummy_dst, nchunk):
    e = ei.shape[1]
    tot = _NW * nchunk * _CHUNK
    pad = tot - e
    src = jnp.concatenate([ei[0], jnp.zeros((pad,), jnp.int32)])
    dst = jnp.concatenate([ei[1], jnp.full((pad,), n_dummy_dst, jnp.int32)])
    return (src.reshape(_NW, nchunk, _CHUNK),
            dst.reshape(_NW, nchunk, _CHUNK))


def _gin_mlp_tc(x, p0, p1, mp, relu_out):
    n = x.shape[0]
    chid = mp["lin0"]["W"].shape[1]
    cout = mp["lin1"]["W"].shape[1]
    br = 256
    grid = pl.cdiv(n, br)

    def body(x_ref, p0_ref, p1_ref,
             w0_ref, b0_ref, w1_ref, b1_ref,
             w2_ref, b2_ref, w3_ref, b3_ref, o_ref):
        h = x_ref[...] + p0_ref[...] + p1_ref[...]
        o1 = jnp.maximum(
            jnp.dot(h, w0_ref[...], preferred_element_type=jnp.float32)
            + b0_ref[...], 0.0)
        o1 = jnp.dot(o1, w1_ref[...],
                     preferred_element_type=jnp.float32) + b1_ref[...]
        o = jnp.maximum(
            jnp.dot(o1, w2_ref[...], preferred_element_type=jnp.float32)
            + b2_ref[...], 0.0)
        o = jnp.dot(o, w3_ref[...],
                    preferred_element_type=jnp.float32) + b3_ref[...] + o1
        if relu_out:
            o = jnp.maximum(o, 0.0)
        o_ref[...] = o

    full = lambda r, c: pl.BlockSpec((r, c), lambda i: (0, 0))
    out = pl.pallas_call(
        body,
        grid=(grid,),
        in_specs=[
            pl.BlockSpec((br, _H), lambda i: (i, 0)),
            pl.BlockSpec((br, _H), lambda i: (i, 0)),
            pl.BlockSpec((br, _H), lambda i: (i, 0)),
            full(_H, chid), full(1, chid),
            full(chid, cout), full(1, cout),
            full(cout, chid), full(1, chid),
            full(chid, cout), full(1, cout),
        ],
        out_specs=pl.BlockSpec((br, cout), lambda i: (i, 0)),
        out_shape=jax.ShapeDtypeStruct((n, cout), jnp.float32),
    )(x, p0, p1,
      mp["lin0"]["W"], mp["lin0"]["b"].reshape(1, -1),
      mp["lin1"]["W"], mp["lin1"]["b"].reshape(1, -1),
      mp["lin2"]["W"], mp["lin2"]["b"].reshape(1, -1),
      mp["lin3"]["W"], mp["lin3"]["b"].reshape(1, -1))
    return out


def _gin_conv(x, src3, dst3, zeros_tile, rows_acc, nchunk, gk, mp, relu_out):
    n = x.shape[0]
    partials = _segment_sum_sc(x, src3, dst3, zeros_tile, rows_acc, nchunk, gk)
    return _gin_mlp_tc(x, partials[0, :n], partials[1, :n], mp, relu_out)


def _knn_interpolate_tc(x, pos_y, pos_xt, k):
    ny = pos_y.shape[0]
    nxp = pos_xt.shape[1]
    h = x.shape[1]
    by = 256
    tile = 1024 if nxp % 1024 == 0 else 512
    nt = nxp // tile
    grid = pl.cdiv(ny, by)
    jbits = max(1, (nxp - 1).bit_length())

    def body(py_ref, pxt_ref, x_ref, o_ref, d_ref):
        yb = py_ref[...]
        y0 = yb[:, 0:1]
        y1 = yb[:, 1:2]
        y2 = yb[:, 2:3]

        def dist_body(t, carry):
            sl = pl.ds(t * tile, tile)
            x0 = pxt_ref[0:1, sl]
            x1 = pxt_ref[1:2, sl]
            x2 = pxt_ref[2:3, sl]
            d = (y0 - x0) ** 2 + (y1 - x1) ** 2 + (y2 - x2) ** 2
            d_ref[:, sl] = d
            return carry

        lax.fori_loop(0, nt, dist_body, 0, unroll=False)

        nsl = tile // 128

        def _lane_fold(m):
            s = m[:, 0:128]
            for q in range(1, nsl):
                s = s + m[:, q * 128:(q + 1) * 128]
            return s

        def count_le(mid):
            def tb(t, acc):
                d = d_ref[:, pl.ds(t * tile, tile)]
                db = lax.bitcast_convert_type(d, jnp.int32)
                return acc + _lane_fold((db <= mid).astype(jnp.int32))
            acc = lax.fori_loop(0, nt, tb, jnp.zeros((by, 128), jnp.int32),
                                unroll=False)
            return jnp.sum(acc, axis=1, keepdims=True)

        def bs_body(_, lohi):
            lo, hi = lohi
            mid = lax.div(lo + hi, 2)
            c = count_le(mid)
            ge = c >= k
            return (jnp.where(ge, lo, mid + 1), jnp.where(ge, mid, hi))

        lo0 = jnp.zeros((by, 1), jnp.int32)
        hi0 = jnp.full((by, 1), jnp.int32(0x7F800000))
        lo, hi = lax.fori_loop(0, 31, bs_body, (lo0, hi0), unroll=False)
        thresh = hi

        def tb_lt_le(t, accs):
            alt, ale = accs
            d = d_ref[:, pl.ds(t * tile, tile)]
            db = lax.bitcast_convert_type(d, jnp.int32)
            return (alt + _lane_fold((db < thresh).astype(jnp.int32)),
                    ale + _lane_fold((db <= thresh).astype(jnp.int32)))

        z128 = jnp.zeros((by, 128), jnp.int32)
        alt, ale = lax.fori_loop(0, nt, tb_lt_le, (z128, z128), unroll=False)
        c_lt = jnp.sum(alt, axis=1, keepdims=True)
        c_le = jnp.sum(ale, axis=1, keepdims=True)
        r = k - c_lt

        def count_eq_le(jcol):
            def tb(t, acc):
                d = d_ref[:, pl.ds(t * tile, tile)]
                db = lax.bitcast_convert_type(d, jnp.int32)
                col = lax.broadcasted_iota(jnp.int32, (by, tile), 1) + t * tile
                m = (db == thresh) & (col <= jcol)
                return acc + _lane_fold(m.astype(jnp.int32))
            acc = lax.fori_loop(0, nt, tb, jnp.zeros((by, 128), jnp.int32),
                                unroll=False)
            return jnp.sum(acc, axis=1, keepdims=True)

        def bs2_body(_, lohi):
            lo, hi = lohi
            mid = lax.div(lo + hi, 2)
            ge = count_eq_le(mid) >= r
            return (jnp.where(ge, lo, mid + 1), jnp.where(ge, mid, hi))

        def eq_search():
            lo0 = jnp.zeros((by, 1), jnp.int32)
            hi0 = jnp.full((by, 1), jnp.int32(nxp - 1))
            _, js = lax.fori_loop(0, jbits, bs2_body, (lo0, hi0),
                                  unroll=False)
            return js

        jstar = lax.cond(jnp.any(c_le - c_lt > r), eq_search,
                         lambda: jnp.full((by, 1), jnp.int32(nxp - 1)))

        def w_body(t, carry):
            sl = pl.ds(t * tile, tile)
            d = d_ref[:, sl]
            db = lax.bitcast_convert_type(d, jnp.int32)
            col = lax.broadcasted_iota(jnp.int32, (by, tile), 1) + t * tile
            sel = (db < thresh) | ((db == thresh) & (col <= jstar))
            w = jnp.where(sel, 1.0 / jnp.maximum(d, 1e-16), 0.0)
            d_ref[:, sl] = w
            return carry

        lax.fori_loop(0, nt, w_body, 0, unroll=False)

        w = d_ref[...]
        num = jnp.dot(w, x_ref[...], preferred_element_type=jnp.float32)
        den = jnp.sum(w, axis=1, keepdims=True)
        o_ref[...] = num / den

    out = pl.pallas_call(
        body,
        grid=(grid,),
        in_specs=[
            pl.BlockSpec((by, 3), lambda i: (i, 0)),
            pl.BlockSpec((3, nxp), lambda i: (0, 0)),
            pl.BlockSpec((nxp, h), lambda i: (0, 0)),
        ],
        out_specs=pl.BlockSpec((by, h), lambda i: (i, 0)),
        out_shape=jax.ShapeDtypeStruct((ny, h), jnp.float32),
        scratch_shapes=[pltpu.VMEM((by, nxp), jnp.float32)],
    )(pos_y, pos_xt, x)
    return out


def _pad_pos(pos, nxp):
    n = pos.shape[0]
    pt = jnp.transpose(pos)
    return jnp.pad(pt, ((0, 0), (0, nxp - n)), constant_values=1e9)


def kernel(latent, edge_index_0, edge_index_1, edge_index_2,
           pos_0, pos_1, pos_2, params):
    p = params

    rows0, nch0 = 2560, 20
    rows1, nch1 = 5120, 40
    rows2, nch2 = 10112, 80
    z0 = jnp.zeros((rows0 // 16, _H), jnp.float32)
    z1 = jnp.zeros((rows1 // 16, _H), jnp.float32)
    z2 = jnp.zeros((rows2 // 16, _H), jnp.float32)
    s0, d0 = _pad_edges(edge_index_0, _N0, nch0)
    s1, d1 = _pad_edges(edge_index_1, _N1, nch1)
    s2, d2 = _pad_edges(edge_index_2, _N2, nch2)

    x = _matvec_head(latent, p["lin_0"]["W"], p["lin_0"]["b"],
                     p["lin_1"]["W"], p["lin_1"]["b"])

    x = _gin_conv(x, s0, d0, z0, rows0, nch0, 2, p["conv0"], True)
    x = _gin_conv(x, s0, d0, z0, rows0, nch0, 2, p["conv1"], True)

    nxp0 = 2560
    xp = jnp.pad(x, ((0, nxp0 - _N0), (0, 0)))
    x = _knn_interpolate_tc(xp, pos_1, _pad_pos(pos_0, nxp0), 32)

    x = _gin_conv(x, s1, d1, z1, rows1, nch1, 4, p["conv2"], True)

    nxp1 = 5120
    xp = jnp.pad(x, ((0, nxp1 - _N1), (0, 0)))
    x = _knn_interpolate_tc(xp, pos_2, _pad_pos(pos_1, nxp1), 32)

    x = _gin_conv(x, s2, d2, z2, rows2, nch2, 4, p["conv3"], True)
    out = _gin_conv(x, s2, d2, z2, rows2, nch2, 4, p["conv4"], False)
    return out
